# Initial kernel scaffold; baseline (speedup 1.0000x reference)
import jax
import jax.numpy as jnp
from jax.experimental import pallas as pl


def kernel(T, bias):
    raise NotImplementedError("write your pallas kernel here")


# speedup vs baseline: 188.4766x; 188.4766x over previous
import math

import jax
import jax.numpy as jnp
from jax.experimental import pallas as pl
from jax.experimental.pallas import tpu as pltpu

_H = 16
_TS = 2048
_NB = 32
_WIDTH = 4352
_BASE = 2304
_LOG_SCALE = 8.0 / math.log(128.0 / 8.0)


def _bias_tc_kernel(bias_smem, out_ref, w_ref):
    h = pl.program_id(0)
    k = jax.lax.broadcasted_iota(jnp.int32, (8, _WIDTH), 1)
    s = jax.lax.broadcasted_iota(jnp.int32, (8, _WIDTH), 0)
    d = k - s - _BASE
    n = jnp.abs(d)
    big = 8 + (jnp.log(n.astype(jnp.float32) * 0.125 + 1e-6) * _LOG_SCALE).astype(jnp.int32)
    big = jnp.minimum(big, 15)
    bucket = jnp.where(n < 8, n, big) + jnp.where(d > 0, 16, 0)
    acc = jnp.zeros((8, _WIDTH), jnp.float32)
    for b in range(_NB):
        acc = jnp.where(bucket == b, bias_smem[b, h], acc)
    w_ref[0:8, :] = acc
    cur = 8
    while cur < 128:
        w_ref[pl.ds(cur, cur), pl.ds(cur, _WIDTH - cur)] = (
            w_ref[pl.ds(0, cur), pl.ds(0, _WIDTH - cur)]
        )
        cur *= 2
    for g in range(16):
        out_ref[0, 0, pl.ds(128 * g, 128), :] = w_ref[:, pl.ds(_BASE - 128 * g, _TS)]


def kernel(T, bias):
    del T
    return pl.pallas_call(
        _bias_tc_kernel,
        grid=(_H,),
        in_specs=[pl.BlockSpec(memory_space=pltpu.SMEM)],
        out_specs=pl.BlockSpec((1, 1, _TS, _TS), lambda h: (0, h, 0, 0)),
        out_shape=jax.ShapeDtypeStruct((1, _H, _TS, _TS), jnp.float32),
        scratch_shapes=[pltpu.VMEM((128, _WIDTH), jnp.float32)],
    )(bias)
